# baseline (speedup 1.0000x reference)
import jax
import jax.numpy as jnp
from jax.experimental import pallas as pl

N = 4096
BM = 256


def _split(y):
    hi = y.astype(jnp.bfloat16)
    lo = (y - hi.astype(jnp.float32)).astype(jnp.bfloat16)
    return hi, lo


def _prep_kernel(adj_ref, feat_ref, feata_ref, w1_ref,
                 b_ref, dinv_ref, y1hi_ref, y1lo_ref):
    i = pl.program_id(0)
    adj = adj_ref[...]
    rows = jax.lax.broadcasted_iota(jnp.int32, adj.shape, 0) + i * BM
    cols = jax.lax.broadcasted_iota(jnp.int32, adj.shape, 1)
    b = jnp.where(cols == rows, 1.0, (adj != 0).astype(jnp.float32))
    b_ref[...] = b.astype(jnp.bfloat16)
    deg = jnp.sum(b, axis=1, keepdims=True)
    dinv = jax.lax.rsqrt(deg)
    dinv_ref[...] = dinv
    xw = jnp.dot(feat_ref[...], w1_ref[...], preferred_element_type=jnp.float32)
    xwa = jnp.dot(feata_ref[...], w1_ref[...], preferred_element_type=jnp.float32)
    y1s = jnp.concatenate([xw, xwa], axis=1) * dinv
    hi, lo = _split(y1s)
    y1hi_ref[...] = hi
    y1lo_ref[...] = lo


def _layer1_kernel(b_ref, y1hi_ref, y1lo_ref, dinv_ref, b1_ref, w2_ref,
                   y2hi_ref, y2lo_ref):
    b = b_ref[...]
    h = (jnp.dot(b, y1hi_ref[...], preferred_element_type=jnp.float32)
         + jnp.dot(b, y1lo_ref[...], preferred_element_type=jnp.float32))
    dinv = dinv_ref[...]
    z = jax.nn.relu(h * dinv + b1_ref[...])
    hdim = w2_ref.shape[0]
    y2 = jnp.dot(z[:, :hdim], w2_ref[...], preferred_element_type=jnp.float32)
    y2a = jnp.dot(z[:, hdim:], w2_ref[...], preferred_element_type=jnp.float32)
    hi, lo = _split(jnp.concatenate([y2, y2a], axis=1) * dinv)
    y2hi_ref[...] = hi
    y2lo_ref[...] = lo


def _layer2_kernel(b_ref, y2hi_ref, y2lo_ref, dinv_ref, b2_ref,
                   emb_ref, ehi_ref, elo_ref):
    b = b_ref[...]
    h = (jnp.dot(b, y2hi_ref[...], preferred_element_type=jnp.float32)
         + jnp.dot(b, y2lo_ref[...], preferred_element_type=jnp.float32))
    emb = h * dinv_ref[...] + b2_ref[...]
    emb_ref[...] = emb
    hi, lo = _split(emb)
    ehi_ref[...] = hi
    elo_ref[...] = lo


def _readout_kernel(g_ref, ehi_ref, elo_ref, embblk_ref, wb_ref, bb_ref,
                    ret_ref, reta_ref):
    gm = g_ref[...]
    gb = gm.astype(jnp.bfloat16)
    vsum = (jnp.dot(gb, ehi_ref[...], preferred_element_type=jnp.float32)
            + jnp.dot(gb, elo_ref[...], preferred_element_type=jnp.float32))
    rs = jnp.sum(gm, axis=1, keepdims=True)
    ge = vsum / rs
    d = ge.shape[1] // 2
    ge1, ge2 = ge[:, :d], ge[:, d:]
    n1 = jnp.sqrt(jnp.sum(ge1 * ge1, axis=1, keepdims=True))
    n2 = jnp.sqrt(jnp.sum(ge2 * ge2, axis=1, keepdims=True))
    g = jax.nn.sigmoid(ge1 / jnp.maximum(n1, 1e-12))
    ga = jax.nn.sigmoid(ge2 / jnp.maximum(n2, 1e-12))
    embblk = embblk_ref[...]
    emb, emba = embblk[:, :d], embblk[:, d:]
    t = jnp.dot(emb, wb_ref[...], preferred_element_type=jnp.float32)
    ta = jnp.dot(emba, wb_ref[...], preferred_element_type=jnp.float32)
    bb = bb_ref[0, 0]
    sc1 = jnp.sum(t * g, axis=1, keepdims=True) + bb
    sc2 = jnp.sum(ta * g, axis=1, keepdims=True) + bb
    sa1 = jnp.sum(ta * ga, axis=1, keepdims=True) + bb
    sa2 = jnp.sum(t * ga, axis=1, keepdims=True) + bb
    ret_ref[...] = jnp.concatenate([sc1, sc2], axis=1)
    reta_ref[...] = jnp.concatenate([sa1, sa2], axis=1)


@jax.jit
def kernel(feat, feat_a, adj, graph_neigh, W1, b1, W2, b2, Wb, bb):
    nblk = N // BM
    hidden = W1.shape[1]
    out_dim = W2.shape[1]

    row_blk = lambda i: (i, 0)
    fixed = lambda i: (0, 0)

    b_bf, dinv, y1hi, y1lo = pl.pallas_call(
        _prep_kernel,
        grid=(nblk,),
        in_specs=[
            pl.BlockSpec((BM, N), row_blk),
            pl.BlockSpec((BM, feat.shape[1]), row_blk),
            pl.BlockSpec((BM, feat.shape[1]), row_blk),
            pl.BlockSpec(W1.shape, fixed),
        ],
        out_specs=[
            pl.BlockSpec((BM, N), row_blk),
            pl.BlockSpec((BM, 1), row_blk),
            pl.BlockSpec((BM, 2 * hidden), row_blk),
            pl.BlockSpec((BM, 2 * hidden), row_blk),
        ],
        out_shape=[
            jax.ShapeDtypeStruct((N, N), jnp.bfloat16),
            jax.ShapeDtypeStruct((N, 1), jnp.float32),
            jax.ShapeDtypeStruct((N, 2 * hidden), jnp.bfloat16),
            jax.ShapeDtypeStruct((N, 2 * hidden), jnp.bfloat16),
        ],
    )(adj, feat, feat_a, W1)

    b1c = jnp.concatenate([b1, b1]).reshape(1, 2 * hidden)
    y2hi, y2lo = pl.pallas_call(
        _layer1_kernel,
        grid=(nblk,),
        in_specs=[
            pl.BlockSpec((BM, N), row_blk),
            pl.BlockSpec((N, 2 * hidden), fixed),
            pl.BlockSpec((N, 2 * hidden), fixed),
            pl.BlockSpec((BM, 1), row_blk),
            pl.BlockSpec((1, 2 * hidden), fixed),
            pl.BlockSpec(W2.shape, fixed),
        ],
        out_specs=[
            pl.BlockSpec((BM, 2 * out_dim), row_blk),
            pl.BlockSpec((BM, 2 * out_dim), row_blk),
        ],
        out_shape=[
            jax.ShapeDtypeStruct((N, 2 * out_dim), jnp.bfloat16),
            jax.ShapeDtypeStruct((N, 2 * out_dim), jnp.bfloat16),
        ],
    )(b_bf, y1hi, y1lo, dinv, b1c, W2)

    b2c = jnp.concatenate([b2, b2]).reshape(1, 2 * out_dim)
    embcat, ehi, elo = pl.pallas_call(
        _layer2_kernel,
        grid=(nblk,),
        in_specs=[
            pl.BlockSpec((BM, N), row_blk),
            pl.BlockSpec((N, 2 * out_dim), fixed),
            pl.BlockSpec((N, 2 * out_dim), fixed),
            pl.BlockSpec((BM, 1), row_blk),
            pl.BlockSpec((1, 2 * out_dim), fixed),
        ],
        out_specs=[
            pl.BlockSpec((BM, 2 * out_dim), row_blk),
            pl.BlockSpec((BM, 2 * out_dim), row_blk),
            pl.BlockSpec((BM, 2 * out_dim), row_blk),
        ],
        out_shape=[
            jax.ShapeDtypeStruct((N, 2 * out_dim), jnp.float32),
            jax.ShapeDtypeStruct((N, 2 * out_dim), jnp.bfloat16),
            jax.ShapeDtypeStruct((N, 2 * out_dim), jnp.bfloat16),
        ],
    )(b_bf, y2hi, y2lo, dinv, b2c)

    ret, ret_a = pl.pallas_call(
        _readout_kernel,
        grid=(nblk,),
        in_specs=[
            pl.BlockSpec((BM, N), row_blk),
            pl.BlockSpec((N, 2 * out_dim), fixed),
            pl.BlockSpec((N, 2 * out_dim), fixed),
            pl.BlockSpec((BM, 2 * out_dim), row_blk),
            pl.BlockSpec(Wb.shape, fixed),
            pl.BlockSpec((1, 1), fixed),
        ],
        out_specs=[
            pl.BlockSpec((BM, 2), row_blk),
            pl.BlockSpec((BM, 2), row_blk),
        ],
        out_shape=[
            jax.ShapeDtypeStruct((N, 2), jnp.float32),
            jax.ShapeDtypeStruct((N, 2), jnp.float32),
        ],
    )(graph_neigh, ehi, elo, embcat, Wb, bb.reshape(1, 1))

    emb = embcat[:, :out_dim]
    return (emb, ret, ret_a)


# speedup vs baseline: 1.1950x; 1.1950x over previous
import jax
import jax.numpy as jnp
from jax.experimental import pallas as pl

N = 4096
BM = 256


def _prep_kernel(adj_ref, feat_ref, feata_ref, w1_ref,
                 b_ref, dinv_ref, y1s_ref):
    i = pl.program_id(0)
    adj = adj_ref[...]
    rows = jax.lax.broadcasted_iota(jnp.int32, adj.shape, 0) + i * BM
    cols = jax.lax.broadcasted_iota(jnp.int32, adj.shape, 1)
    b = jnp.where(cols == rows, 1.0, (adj != 0).astype(jnp.float32))
    b_ref[...] = b.astype(jnp.bfloat16)
    deg = jnp.sum(b, axis=1, keepdims=True)
    dinv = jax.lax.rsqrt(deg)
    dinv_ref[...] = dinv
    xw = jnp.dot(feat_ref[...], w1_ref[...], preferred_element_type=jnp.float32)
    xwa = jnp.dot(feata_ref[...], w1_ref[...], preferred_element_type=jnp.float32)
    y1s_ref[...] = (jnp.concatenate([xw, xwa], axis=1) * dinv).astype(jnp.bfloat16)


def _layer1_kernel(b_ref, y1s_ref, dinv_ref, b1_ref, w2_ref, y2s_ref):
    h = jnp.dot(b_ref[...], y1s_ref[...], preferred_element_type=jnp.float32)
    dinv = dinv_ref[...]
    z = jax.nn.relu(h * dinv + b1_ref[...])
    hdim = w2_ref.shape[0]
    y2 = jnp.dot(z[:, :hdim], w2_ref[...], preferred_element_type=jnp.float32)
    y2a = jnp.dot(z[:, hdim:], w2_ref[...], preferred_element_type=jnp.float32)
    y2s_ref[...] = (jnp.concatenate([y2, y2a], axis=1) * dinv).astype(jnp.bfloat16)


def _layer2_kernel(b_ref, y2s_ref, dinv_ref, b2_ref, emb_ref, ebf_ref):
    h = jnp.dot(b_ref[...], y2s_ref[...], preferred_element_type=jnp.float32)
    emb = h * dinv_ref[...] + b2_ref[...]
    emb_ref[...] = emb
    ebf_ref[...] = emb.astype(jnp.bfloat16)


def _readout_kernel(g_ref, ebf_ref, embblk_ref, wb_ref, bb_ref,
                    ret_ref, reta_ref):
    gm = g_ref[...]
    gb = gm.astype(jnp.bfloat16)
    vsum = jnp.dot(gb, ebf_ref[...], preferred_element_type=jnp.float32)
    rs = jnp.sum(gm, axis=1, keepdims=True)
    ge = vsum / rs
    d = ge.shape[1] // 2
    ge1, ge2 = ge[:, :d], ge[:, d:]
    n1 = jnp.sqrt(jnp.sum(ge1 * ge1, axis=1, keepdims=True))
    n2 = jnp.sqrt(jnp.sum(ge2 * ge2, axis=1, keepdims=True))
    g = jax.nn.sigmoid(ge1 / jnp.maximum(n1, 1e-12))
    ga = jax.nn.sigmoid(ge2 / jnp.maximum(n2, 1e-12))
    embblk = embblk_ref[...]
    emb, emba = embblk[:, :d], embblk[:, d:]
    t = jnp.dot(emb, wb_ref[...], preferred_element_type=jnp.float32)
    ta = jnp.dot(emba, wb_ref[...], preferred_element_type=jnp.float32)
    bb = bb_ref[0, 0]
    sc1 = jnp.sum(t * g, axis=1, keepdims=True) + bb
    sc2 = jnp.sum(ta * g, axis=1, keepdims=True) + bb
    sa1 = jnp.sum(ta * ga, axis=1, keepdims=True) + bb
    sa2 = jnp.sum(t * ga, axis=1, keepdims=True) + bb
    ret_ref[...] = jnp.concatenate([sc1, sc2], axis=1)
    reta_ref[...] = jnp.concatenate([sa1, sa2], axis=1)


@jax.jit
def kernel(feat, feat_a, adj, graph_neigh, W1, b1, W2, b2, Wb, bb):
    nblk = N // BM
    hidden = W1.shape[1]
    out_dim = W2.shape[1]

    row_blk = lambda i: (i, 0)
    fixed = lambda i: (0, 0)

    b_bf, dinv, y1s = pl.pallas_call(
        _prep_kernel,
        grid=(nblk,),
        in_specs=[
            pl.BlockSpec((BM, N), row_blk),
            pl.BlockSpec((BM, feat.shape[1]), row_blk),
            pl.BlockSpec((BM, feat.shape[1]), row_blk),
            pl.BlockSpec(W1.shape, fixed),
        ],
        out_specs=[
            pl.BlockSpec((BM, N), row_blk),
            pl.BlockSpec((BM, 1), row_blk),
            pl.BlockSpec((BM, 2 * hidden), row_blk),
        ],
        out_shape=[
            jax.ShapeDtypeStruct((N, N), jnp.bfloat16),
            jax.ShapeDtypeStruct((N, 1), jnp.float32),
            jax.ShapeDtypeStruct((N, 2 * hidden), jnp.bfloat16),
        ],
    )(adj, feat, feat_a, W1)

    b1c = jnp.concatenate([b1, b1]).reshape(1, 2 * hidden)
    y2s = pl.pallas_call(
        _layer1_kernel,
        grid=(nblk,),
        in_specs=[
            pl.BlockSpec((BM, N), row_blk),
            pl.BlockSpec((N, 2 * hidden), fixed),
            pl.BlockSpec((BM, 1), row_blk),
            pl.BlockSpec((1, 2 * hidden), fixed),
            pl.BlockSpec(W2.shape, fixed),
        ],
        out_specs=pl.BlockSpec((BM, 2 * out_dim), row_blk),
        out_shape=jax.ShapeDtypeStruct((N, 2 * out_dim), jnp.bfloat16),
    )(b_bf, y1s, dinv, b1c, W2)

    b2c = jnp.concatenate([b2, b2]).reshape(1, 2 * out_dim)
    embcat, ebf = pl.pallas_call(
        _layer2_kernel,
        grid=(nblk,),
        in_specs=[
            pl.BlockSpec((BM, N), row_blk),
            pl.BlockSpec((N, 2 * out_dim), fixed),
            pl.BlockSpec((BM, 1), row_blk),
            pl.BlockSpec((1, 2 * out_dim), fixed),
        ],
        out_specs=[
            pl.BlockSpec((BM, 2 * out_dim), row_blk),
            pl.BlockSpec((BM, 2 * out_dim), row_blk),
        ],
        out_shape=[
            jax.ShapeDtypeStruct((N, 2 * out_dim), jnp.float32),
            jax.ShapeDtypeStruct((N, 2 * out_dim), jnp.bfloat16),
        ],
    )(b_bf, y2s, dinv, b2c)

    ret, ret_a = pl.pallas_call(
        _readout_kernel,
        grid=(nblk,),
        in_specs=[
            pl.BlockSpec((BM, N), row_blk),
            pl.BlockSpec((N, 2 * out_dim), fixed),
            pl.BlockSpec((BM, 2 * out_dim), row_blk),
            pl.BlockSpec(Wb.shape, fixed),
            pl.BlockSpec((1, 1), fixed),
        ],
        out_specs=[
            pl.BlockSpec((BM, 2), row_blk),
            pl.BlockSpec((BM, 2), row_blk),
        ],
        out_shape=[
            jax.ShapeDtypeStruct((N, 2), jnp.float32),
            jax.ShapeDtypeStruct((N, 2), jnp.float32),
        ],
    )(graph_neigh, ebf, embcat, Wb, bb.reshape(1, 1))

    emb = embcat[:, :out_dim]
    return (emb, ret, ret_a)


# speedup vs baseline: 1.2696x; 1.0624x over previous
import jax
import jax.numpy as jnp
from jax.experimental import pallas as pl

N = 4096
BM = 256


def _prep_kernel(adj_ref, feat_ref, feata_ref, w1_ref,
                 b_ref, dinv_ref, y1s_ref):
    i = pl.program_id(0)
    adj = adj_ref[...]
    rows = jax.lax.broadcasted_iota(jnp.int32, adj.shape, 0) + i * BM
    cols = jax.lax.broadcasted_iota(jnp.int32, adj.shape, 1)
    b = jnp.where(cols == rows, 1.0, (adj != 0).astype(jnp.float32))
    b_ref[...] = b.astype(jnp.int8)
    deg = jnp.sum(b, axis=1, keepdims=True)
    dinv = jax.lax.rsqrt(deg)
    dinv_ref[...] = dinv
    xw = jnp.dot(feat_ref[...], w1_ref[...], preferred_element_type=jnp.float32)
    xwa = jnp.dot(feata_ref[...], w1_ref[...], preferred_element_type=jnp.float32)
    y1s_ref[...] = (jnp.concatenate([xw, xwa], axis=1) * dinv).astype(jnp.bfloat16)


def _layer1_kernel(b_ref, y1s_ref, dinv_ref, b1_ref, w2_ref, y2s_ref):
    h = jnp.dot(b_ref[...].astype(jnp.bfloat16), y1s_ref[...],
                preferred_element_type=jnp.float32)
    dinv = dinv_ref[...]
    z = jax.nn.relu(h * dinv + b1_ref[...])
    hdim = w2_ref.shape[0]
    y2 = jnp.dot(z[:, :hdim], w2_ref[...], preferred_element_type=jnp.float32)
    y2a = jnp.dot(z[:, hdim:], w2_ref[...], preferred_element_type=jnp.float32)
    y2s_ref[...] = (jnp.concatenate([y2, y2a], axis=1) * dinv).astype(jnp.bfloat16)


def _layer2_kernel(b_ref, y2s_ref, dinv_ref, b2_ref, emb_ref, ebf_ref):
    h = jnp.dot(b_ref[...].astype(jnp.bfloat16), y2s_ref[...],
                preferred_element_type=jnp.float32)
    emb = h * dinv_ref[...] + b2_ref[...]
    emb_ref[...] = emb
    ebf_ref[...] = emb.astype(jnp.bfloat16)


def _readout_kernel(g_ref, ebf_ref, embblk_ref, wb_ref, bb_ref,
                    ret_ref, reta_ref):
    gm = g_ref[...]
    gb = gm.astype(jnp.bfloat16)
    vsum = jnp.dot(gb, ebf_ref[...], preferred_element_type=jnp.float32)
    rs = jnp.sum(gm, axis=1, keepdims=True)
    ge = vsum / rs
    d = ge.shape[1] // 2
    ge1, ge2 = ge[:, :d], ge[:, d:]
    n1 = jnp.sqrt(jnp.sum(ge1 * ge1, axis=1, keepdims=True))
    n2 = jnp.sqrt(jnp.sum(ge2 * ge2, axis=1, keepdims=True))
    g = jax.nn.sigmoid(ge1 / jnp.maximum(n1, 1e-12))
    ga = jax.nn.sigmoid(ge2 / jnp.maximum(n2, 1e-12))
    embblk = embblk_ref[...]
    emb, emba = embblk[:, :d], embblk[:, d:]
    t = jnp.dot(emb, wb_ref[...], preferred_element_type=jnp.float32)
    ta = jnp.dot(emba, wb_ref[...], preferred_element_type=jnp.float32)
    bb = bb_ref[0, 0]
    sc1 = jnp.sum(t * g, axis=1, keepdims=True) + bb
    sc2 = jnp.sum(ta * g, axis=1, keepdims=True) + bb
    sa1 = jnp.sum(ta * ga, axis=1, keepdims=True) + bb
    sa2 = jnp.sum(t * ga, axis=1, keepdims=True) + bb
    ret_ref[...] = jnp.concatenate([sc1, sc2], axis=1)
    reta_ref[...] = jnp.concatenate([sa1, sa2], axis=1)


@jax.jit
def kernel(feat, feat_a, adj, graph_neigh, W1, b1, W2, b2, Wb, bb):
    nblk = N // BM
    hidden = W1.shape[1]
    out_dim = W2.shape[1]

    row_blk = lambda i: (i, 0)
    fixed = lambda i: (0, 0)

    b_bf, dinv, y1s = pl.pallas_call(
        _prep_kernel,
        grid=(nblk,),
        in_specs=[
            pl.BlockSpec((BM, N), row_blk),
            pl.BlockSpec((BM, feat.shape[1]), row_blk),
            pl.BlockSpec((BM, feat.shape[1]), row_blk),
            pl.BlockSpec(W1.shape, fixed),
        ],
        out_specs=[
            pl.BlockSpec((BM, N), row_blk),
            pl.BlockSpec((BM, 1), row_blk),
            pl.BlockSpec((BM, 2 * hidden), row_blk),
        ],
        out_shape=[
            jax.ShapeDtypeStruct((N, N), jnp.int8),
            jax.ShapeDtypeStruct((N, 1), jnp.float32),
            jax.ShapeDtypeStruct((N, 2 * hidden), jnp.bfloat16),
        ],
    )(adj, feat, feat_a, W1)

    b1c = jnp.concatenate([b1, b1]).reshape(1, 2 * hidden)
    y2s = pl.pallas_call(
        _layer1_kernel,
        grid=(nblk,),
        in_specs=[
            pl.BlockSpec((BM, N), row_blk),
            pl.BlockSpec((N, 2 * hidden), fixed),
            pl.BlockSpec((BM, 1), row_blk),
            pl.BlockSpec((1, 2 * hidden), fixed),
            pl.BlockSpec(W2.shape, fixed),
        ],
        out_specs=pl.BlockSpec((BM, 2 * out_dim), row_blk),
        out_shape=jax.ShapeDtypeStruct((N, 2 * out_dim), jnp.bfloat16),
    )(b_bf, y1s, dinv, b1c, W2)

    b2c = jnp.concatenate([b2, b2]).reshape(1, 2 * out_dim)
    embcat, ebf = pl.pallas_call(
        _layer2_kernel,
        grid=(nblk,),
        in_specs=[
            pl.BlockSpec((BM, N), row_blk),
            pl.BlockSpec((N, 2 * out_dim), fixed),
            pl.BlockSpec((BM, 1), row_blk),
            pl.BlockSpec((1, 2 * out_dim), fixed),
        ],
        out_specs=[
            pl.BlockSpec((BM, 2 * out_dim), row_blk),
            pl.BlockSpec((BM, 2 * out_dim), row_blk),
        ],
        out_shape=[
            jax.ShapeDtypeStruct((N, 2 * out_dim), jnp.float32),
            jax.ShapeDtypeStruct((N, 2 * out_dim), jnp.bfloat16),
        ],
    )(b_bf, y2s, dinv, b2c)

    ret, ret_a = pl.pallas_call(
        _readout_kernel,
        grid=(nblk,),
        in_specs=[
            pl.BlockSpec((BM, N), row_blk),
            pl.BlockSpec((N, 2 * out_dim), fixed),
            pl.BlockSpec((BM, 2 * out_dim), row_blk),
            pl.BlockSpec(Wb.shape, fixed),
            pl.BlockSpec((1, 1), fixed),
        ],
        out_specs=[
            pl.BlockSpec((BM, 2), row_blk),
            pl.BlockSpec((BM, 2), row_blk),
        ],
        out_shape=[
            jax.ShapeDtypeStruct((N, 2), jnp.float32),
            jax.ShapeDtypeStruct((N, 2), jnp.float32),
        ],
    )(graph_neigh, ebf, embcat, Wb, bb.reshape(1, 1))

    emb = embcat[:, :out_dim]
    return (emb, ret, ret_a)
